# baseline (speedup 1.0000x reference)
import functools

import jax
import jax.numpy as jnp
from jax import lax
from jax.experimental import pallas as pl
from jax.experimental.pallas import tpu as pltpu
from jax.experimental.pallas import tpu_sc as plsc

N = 1000000
RHO = 0.99
ALPHA = 0.5
TEMP = 0.5
B = 4096
EPS = 1e-6

NC = 2
NS = 16
NW = NC * NS
CHUNK = B // NW
LANES = 16


def _sc_update_body(s_ref, idx_hbm, v4_hbm, r2_hbm, rep_hbm,
                    idx_v, s_v, v4_v, r2_v, out_v, rep_v, sem):
    wid = lax.axis_index("s") * NC + lax.axis_index("c")
    base = wid * CHUNK
    pltpu.sync_copy(idx_hbm.at[pl.ds(base, CHUNK)], idx_v)
    pltpu.sync_copy(v4_hbm.at[pl.ds(base, CHUNK)], v4_v)
    pltpu.sync_copy(r2_hbm.at[pl.ds(base, CHUNK)], r2_v)
    pltpu.async_copy(s_ref.at[idx_v], s_v, sem).wait()
    scale = jnp.float32((1.0 - RHO) * float(N) * float(N))
    rep_acc = jnp.zeros((LANES,), jnp.float32)
    for j in range(CHUNK // LANES):
        sl = pl.ds(j * LANES, LANES)
        s = s_v[sl]
        out_v[sl] = RHO * s + scale * v4_v[sl]
        rep_acc = rep_acc + r2_v[sl] / s
    rep_v[...] = rep_acc
    pltpu.async_copy(out_v, s_ref.at[idx_v], sem).wait()
    pltpu.sync_copy(rep_v, rep_hbm.at[pl.ds(wid * LANES, LANES)])


def _dense_body(f_ref, attr_ref, v4_ref, r2_ref):
    f = f_ref[...]
    norm = jnp.maximum(jnp.sqrt(jnp.sum(f * f, axis=1, keepdims=True)), 1e-12)
    fn = f / norm
    an = fn[:B]
    bn = fn[B:]
    bro = pltpu.roll(bn, B - 1, 0)
    aro = pltpu.roll(an, B - 1, 0)
    d2aa = jnp.sum((an - bn + EPS) ** 2, axis=1, keepdims=True)
    d2bb = jnp.sum((bn - an + EPS) ** 2, axis=1, keepdims=True)
    d2ra = jnp.sum((an - bro + EPS) ** 2, axis=1, keepdims=True)
    d2rb = jnp.sum((bn - aro + EPS) ** 2, axis=1, keepdims=True)
    inv2t2 = 1.0 / (2.0 * TEMP * TEMP)
    qaa = jnp.exp(-inv2t2 * d2aa)
    qab = jnp.exp(-inv2t2 * d2bb)
    qra = jnp.exp(-inv2t2 * d2ra)
    qrb = jnp.exp(-inv2t2 * d2rb)
    v4_ref[...] = (ALPHA * 0.5) * (qaa + qab) + ((1.0 - ALPHA) * 0.5) * (qra + qrb)
    r2_ref[...] = qra + qrb
    attr_ref[0, 0] = inv2t2 * jnp.sum(d2aa + d2bb)


@functools.cache
def _build():
    mesh = plsc.VectorSubcoreMesh(
        core_axis_name="c", subcore_axis_name="s", num_cores=NC, num_subcores=NS
    )
    sc_update = pl.kernel(
        _sc_update_body,
        out_type=jax.ShapeDtypeStruct((NW * LANES,), jnp.float32),
        mesh=mesh,
        scratch_types=[
            pltpu.VMEM((CHUNK,), jnp.int32),
            pltpu.VMEM((CHUNK,), jnp.float32),
            pltpu.VMEM((CHUNK,), jnp.float32),
            pltpu.VMEM((CHUNK,), jnp.float32),
            pltpu.VMEM((CHUNK,), jnp.float32),
            pltpu.VMEM((LANES,), jnp.float32),
            pltpu.SemaphoreType.DMA,
        ],
    )
    dense = pl.pallas_call(
        _dense_body,
        out_shape=[
            jax.ShapeDtypeStruct((1, 1), jnp.float32),
            jax.ShapeDtypeStruct((B, 1), jnp.float32),
            jax.ShapeDtypeStruct((B, 1), jnp.float32),
        ],
        in_specs=[pl.BlockSpec(memory_space=pltpu.VMEM)],
        out_specs=[
            pl.BlockSpec(memory_space=pltpu.SMEM),
            pl.BlockSpec(memory_space=pltpu.VMEM),
            pl.BlockSpec(memory_space=pltpu.VMEM),
        ],
    )
    return sc_update, dense


def kernel(feats, s_inv, feats_idx):
    sc_update, dense = _build()
    idx = feats_idx.astype(jnp.int32)
    attr2d, v4, r2 = dense(feats)
    s_ref = jax.new_ref(s_inv)
    rep = sc_update(s_ref, idx, v4.reshape(B), r2.reshape(B))
    new_s_inv = s_ref[...]
    n2 = jnp.float32(N) * jnp.float32(N)
    loss = 0.5 * (attr2d[0, 0] + n2 * jnp.sum(rep)) / jnp.float32(B)
    return loss, new_s_inv
